# baseline (speedup 1.0000x reference)
import jax
import jax.numpy as jnp
from jax import lax
from jax.experimental import pallas as pl


def _router_block(fac_ref, x_ref, w_ref, disp_ref, probs_ref, idx_ref):
    f = fac_ref[0, 0]
    x = x_ref[0] * f
    w = w_ref[...]
    lt = jax.lax.dot_general(
        w, x, (((1,), (1,)), ((), ())),
        preferred_element_type=jnp.float32)
    m = jnp.max(lt, axis=0, keepdims=True)
    ex = jnp.exp(lt - m)
    z = jnp.sum(ex, axis=0, keepdims=True)
    probs = ex / z
    probs_ref[0] = probs

    iota = lax.broadcasted_iota(jnp.int32, probs.shape, 0)
    p1 = jnp.max(probs, axis=0, keepdims=True)
    i1 = jnp.min(jnp.where(probs == p1, iota, 64), axis=0, keepdims=True)
    at1 = iota == i1
    probs2 = jnp.where(at1, -1.0, probs)
    p2 = jnp.max(probs2, axis=0, keepdims=True)
    i2 = jnp.min(jnp.where(probs2 == p2, iota, 64), axis=0, keepdims=True)
    s = p1 + p2
    w1 = p1 / s
    w2 = p2 / s
    disp_ref[0] = jnp.where(at1, w1, jnp.where(iota == i2, w2, 0.0))
    idx_ref[0] = jnp.concatenate([i1, i2], axis=0)


def _route(x, w, factor, block_t=4096):
    b, s, d = x.shape
    e = w.shape[0]
    grid = (b, s // block_t)
    return pl.pallas_call(
        _router_block,
        grid=grid,
        in_specs=[
            pl.BlockSpec((1, 1), lambda i, j: (0, 0)),
            pl.BlockSpec((1, block_t, d), lambda i, j: (i, j, 0)),
            pl.BlockSpec((e, d), lambda i, j: (0, 0)),
        ],
        out_specs=[
            pl.BlockSpec((1, e, block_t), lambda i, j: (i, 0, j)),
            pl.BlockSpec((1, e, block_t), lambda i, j: (i, 0, j)),
            pl.BlockSpec((1, 2, block_t), lambda i, j: (i, 0, j)),
        ],
        out_shape=[
            jax.ShapeDtypeStruct((b, e, s), jnp.float32),
            jax.ShapeDtypeStruct((b, e, s), jnp.float32),
            jax.ShapeDtypeStruct((b, 2, s), jnp.int32),
        ],
    )(factor, x, w)


def kernel(x, scale_condition, W, scale_idx):
    factor = (1.0 + scale_condition[scale_idx] * 0.1).reshape(1, 1)
    disp_t, probs_t, idx_t = _route(x, W, factor)
    return (disp_t.transpose(0, 2, 1), probs_t.transpose(0, 2, 1),
            idx_t.transpose(0, 2, 1))


# speedup vs baseline: 1.0046x; 1.0046x over previous
import jax
import jax.numpy as jnp
from jax import lax
from jax.experimental import pallas as pl
from jax.experimental.pallas import tpu as pltpu


def _router_block(fac_ref, x_ref, w_ref, disp_ref, probs_ref, idx_ref):
    f = fac_ref[0, 0]
    x = x_ref[0] * f
    w = w_ref[...]
    lt = jax.lax.dot_general(
        w, x, (((1,), (1,)), ((), ())),
        preferred_element_type=jnp.float32)
    m = jnp.max(lt, axis=0, keepdims=True)
    ex = jnp.exp(lt - m)
    z = jnp.sum(ex, axis=0, keepdims=True)
    probs = ex / z
    probs_ref[0] = probs

    iota = lax.broadcasted_iota(jnp.int32, probs.shape, 0)
    p1 = jnp.max(probs, axis=0, keepdims=True)
    i1 = jnp.min(jnp.where(probs == p1, iota, 64), axis=0, keepdims=True)
    at1 = iota == i1
    probs2 = jnp.where(at1, -1.0, probs)
    p2 = jnp.max(probs2, axis=0, keepdims=True)
    i2 = jnp.min(jnp.where(probs2 == p2, iota, 64), axis=0, keepdims=True)
    s = p1 + p2
    w1 = p1 / s
    w2 = p2 / s
    disp_ref[0] = jnp.where(at1, w1, jnp.where(iota == i2, w2, 0.0))
    idx_ref[0] = jnp.concatenate([i1, i2], axis=0)


def _route(x, w, factor, block_t=4096):
    b, s, d = x.shape
    e = w.shape[0]
    grid = (b, s // block_t)
    return pl.pallas_call(
        _router_block,
        grid=grid,
        compiler_params=pltpu.CompilerParams(
            dimension_semantics=("parallel", "parallel")),
        in_specs=[
            pl.BlockSpec((1, 1), lambda i, j: (0, 0)),
            pl.BlockSpec((1, block_t, d), lambda i, j: (i, j, 0)),
            pl.BlockSpec((e, d), lambda i, j: (0, 0)),
        ],
        out_specs=[
            pl.BlockSpec((1, e, block_t), lambda i, j: (i, 0, j)),
            pl.BlockSpec((1, e, block_t), lambda i, j: (i, 0, j)),
            pl.BlockSpec((1, 2, block_t), lambda i, j: (i, 0, j)),
        ],
        out_shape=[
            jax.ShapeDtypeStruct((b, e, s), jnp.float32),
            jax.ShapeDtypeStruct((b, e, s), jnp.float32),
            jax.ShapeDtypeStruct((b, 2, s), jnp.int32),
        ],
    )(factor, x, w)


def kernel(x, scale_condition, W, scale_idx):
    factor = (1.0 + scale_condition[scale_idx] * 0.1).reshape(1, 1)
    disp_t, probs_t, idx_t = _route(x, W, factor)
    return (disp_t.transpose(0, 2, 1), probs_t.transpose(0, 2, 1),
            idx_t.transpose(0, 2, 1))
